# baseline (speedup 1.0000x reference)
import functools

import jax
import jax.numpy as jnp
from jax import lax
from jax.experimental import pallas as pl
from jax.experimental.pallas import tpu as pltpu
from jax.experimental.pallas import tpu_sc as plsc

VOCAB = 100000
D = 128
ROWS_PER_BLOCK = 16384
NC, NS, LANES = 2, 16, 16
NW = NC * NS
CHUNK = 128


def _fuse_body(gamma_ref, mask_ref, emb_ref, feat_ref, w_ref, out_ref):
    proj = lax.dot_general(
        feat_ref[...], w_ref[...],
        dimension_numbers=(((1,), (1,)), ((), ())),
        preferred_element_type=jnp.float32,
    )
    m = mask_ref[...].astype(jnp.float32)
    scale = gamma_ref[0] * lax.broadcast_in_dim(m, (ROWS_PER_BLOCK, D), (0,))
    out_ref[...] = emb_ref[...] + scale * proj


def _fused_table(gamma, mask1d, id_embed, feat_tbl, feat_proj_w):
    grid = pl.cdiv(VOCAB, ROWS_PER_BLOCK)
    return pl.pallas_call(
        _fuse_body,
        grid=(grid,),
        in_specs=[
            pl.BlockSpec(memory_space=pltpu.SMEM),
            pl.BlockSpec((ROWS_PER_BLOCK,), lambda i: (i,)),
            pl.BlockSpec((ROWS_PER_BLOCK, D), lambda i: (i, 0)),
            pl.BlockSpec((ROWS_PER_BLOCK, D), lambda i: (i, 0)),
            pl.BlockSpec((D, D), lambda i: (0, 0)),
        ],
        out_specs=pl.BlockSpec((ROWS_PER_BLOCK, D), lambda i: (i, 0)),
        out_shape=jax.ShapeDtypeStruct((VOCAB, D), jnp.float32),
    )(gamma, mask1d, id_embed, feat_tbl, feat_proj_w)


NBUF = 5


def _gather_body(seq, bw, tbl_hbm, idx_hbm, out_hbm, idx_v, bufs, gsems,
                 ssems):
    wid = lax.axis_index("s") * NC + lax.axis_index("c")
    c0 = wid * bw
    pltpu.sync_copy(idx_hbm.at[:, pl.ds(c0, bw)], idx_v)

    @pl.loop(0, seq // NBUF)
    def _(p):
        l = NBUF * p
        ghs = [
            pltpu.async_copy(tbl_hbm.at[idx_v.at[l + j]], bufs[j], gsems[j])
            for j in range(NBUF)
        ]
        shs = []
        for j in range(NBUF):
            ghs[j].wait()
            shs.append(
                pltpu.async_copy(bufs[j], out_hbm.at[l + j].at[pl.ds(c0, bw)],
                                 ssems[j]))
        for sh in shs:
            sh.wait()


def _sc_gather(fused, ids_t):
    seq, batch = ids_t.shape
    bw = batch // NW
    mesh = plsc.VectorSubcoreMesh(core_axis_name="c", subcore_axis_name="s")
    return pl.kernel(
        functools.partial(_gather_body, seq, bw),
        out_type=jax.ShapeDtypeStruct((seq, batch, D), jnp.float32),
        mesh=mesh,
        scratch_types=[
            pltpu.VMEM((seq, bw), jnp.int32),
            [pltpu.VMEM((bw, D), jnp.float32) for _ in range(NBUF)],
            [pltpu.SemaphoreType.DMA for _ in range(NBUF)],
            [pltpu.SemaphoreType.DMA for _ in range(NBUF)],
        ],
    )(fused, ids_t)


def kernel(ids, id_embed, feat_tbl, feat_proj_w, prod_mask, gamma):
    gamma1 = gamma.reshape(1).astype(jnp.float32)
    mask1d = prod_mask.astype(jnp.int32)
    fused = _fused_table(gamma1, mask1d, id_embed, feat_tbl, feat_proj_w)
    out_t = _sc_gather(fused, ids.T)
    return out_t.transpose(1, 0, 2)


# speedup vs baseline: 1.0045x; 1.0045x over previous
import functools

import jax
import jax.numpy as jnp
from jax import lax
from jax.experimental import pallas as pl
from jax.experimental.pallas import tpu as pltpu
from jax.experimental.pallas import tpu_sc as plsc

VOCAB = 100000
D = 128
ROWS_PER_BLOCK = 8192
NC, NS, LANES = 2, 16, 16
NW = NC * NS


def _fuse_body(gamma_ref, mask_ref, emb_ref, feat_ref, w_ref, out_ref):
    proj = lax.dot_general(
        feat_ref[...], w_ref[...],
        dimension_numbers=(((1,), (1,)), ((), ())),
        preferred_element_type=jnp.float32,
    )
    m = mask_ref[...].astype(jnp.float32)
    scale = gamma_ref[0] * lax.broadcast_in_dim(m, (ROWS_PER_BLOCK, D), (0,))
    out_ref[...] = emb_ref[...] + scale * proj


def _fused_table(gamma, mask1d, id_embed, feat_tbl, feat_proj_w):
    grid = pl.cdiv(VOCAB, ROWS_PER_BLOCK)
    return pl.pallas_call(
        _fuse_body,
        grid=(grid,),
        in_specs=[
            pl.BlockSpec(memory_space=pltpu.SMEM),
            pl.BlockSpec((ROWS_PER_BLOCK,), lambda i: (i,)),
            pl.BlockSpec((ROWS_PER_BLOCK, D), lambda i: (i, 0)),
            pl.BlockSpec((ROWS_PER_BLOCK, D), lambda i: (i, 0)),
            pl.BlockSpec((D, D), lambda i: (0, 0)),
        ],
        out_specs=pl.BlockSpec((ROWS_PER_BLOCK, D), lambda i: (i, 0)),
        out_shape=jax.ShapeDtypeStruct((VOCAB, D), jnp.float32),
    )(gamma, mask1d, id_embed, feat_tbl, feat_proj_w)


NBUF = 5


def _gather_body(seq, bw, tbl_hbm, idx_hbm, out_hbm, idx_v, bufs, gsems,
                 ssems):
    wid = lax.axis_index("s") * NC + lax.axis_index("c")
    c0 = wid * bw
    pltpu.sync_copy(idx_hbm.at[:, pl.ds(c0, bw)], idx_v)

    @pl.loop(0, seq // NBUF)
    def _(p):
        l = NBUF * p
        ghs = [
            pltpu.async_copy(tbl_hbm.at[idx_v.at[l + j]], bufs[j], gsems[j])
            for j in range(NBUF)
        ]
        shs = []
        for j in range(NBUF):
            ghs[j].wait()
            shs.append(
                pltpu.async_copy(bufs[j], out_hbm.at[l + j].at[pl.ds(c0, bw)],
                                 ssems[j]))
        for sh in shs:
            sh.wait()


def _sc_gather(fused, ids_t):
    seq, batch = ids_t.shape
    bw = batch // NW
    mesh = plsc.VectorSubcoreMesh(core_axis_name="c", subcore_axis_name="s")
    return pl.kernel(
        functools.partial(_gather_body, seq, bw),
        out_type=jax.ShapeDtypeStruct((seq, batch, D), jnp.float32),
        mesh=mesh,
        scratch_types=[
            pltpu.VMEM((seq, bw), jnp.int32),
            [pltpu.VMEM((bw, D), jnp.float32) for _ in range(NBUF)],
            [pltpu.SemaphoreType.DMA for _ in range(NBUF)],
            [pltpu.SemaphoreType.DMA for _ in range(NBUF)],
        ],
    )(fused, ids_t)


def kernel(ids, id_embed, feat_tbl, feat_proj_w, prod_mask, gamma):
    gamma1 = gamma.reshape(1).astype(jnp.float32)
    mask1d = prod_mask.astype(jnp.int32)
    fused = _fused_table(gamma1, mask1d, id_embed, feat_tbl, feat_proj_w)
    out_t = _sc_gather(fused, ids.T)
    return out_t.transpose(1, 0, 2)
